# baseline (device time: 32094 ns/iter reference)
import jax
import jax.numpy as jnp
from jax import lax
from jax.experimental import pallas as pl
from jax.experimental.pallas import tpu as pltpu

BLK = 256
NC = 4
CD = 1024 // NC


def kernel(dy, W):
    m, f = dy.shape
    d = W.shape[0]

    def body(
        dy_hbm, w_hbm, out_ref,
        dyb, a_ref, wbuf,
        psend, precv, rsend, rcv_y, rcv_z, rcv_d,
        dy_sem, w_sems, sA_s, sA_r, sY_s, sY_r, sZ_s, sZ_r, sD_s, sD_r,
    ):
        my_x = lax.axis_index("x")
        my_y = lax.axis_index("y")
        my_z = lax.axis_index("z")
        px = (1 - my_x, my_y, my_z)
        py = (my_x, 1 - my_y, my_z)
        pz = (my_x, my_y, 1 - my_z)
        pd = (my_x, 1 - my_y, 1 - my_z)

        q = 2 * my_y + my_z
        qy = 2 * (1 - my_y) + my_z
        qz = 2 * my_y + (1 - my_z)
        qd = 2 * (1 - my_y) + (1 - my_z)

        barrier = pltpu.get_barrier_semaphore()
        for p in (px, py, pz, pd):
            pl.semaphore_signal(
                barrier, inc=1, device_id=p,
                device_id_type=pl.DeviceIdType.MESH,
            )
        pl.semaphore_wait(barrier, 4)

        dy_cp = pltpu.make_async_copy(
            dy_hbm.at[pl.ds(q * BLK, BLK), :], dyb, dy_sem
        )
        dy_cp.start()

        def w_copy(c):
            return pltpu.make_async_copy(
                w_hbm.at[pl.ds(c * CD, CD), :], wbuf.at[c % 2],
                w_sems.at[c % 2],
            )

        w_copy(0).start()
        dy_cp.wait()
        a_ref[...] = dyb[...].astype(jnp.bfloat16)

        def rdma_a(c):
            return pltpu.make_async_remote_copy(
                src_ref=psend.at[c], dst_ref=precv.at[c],
                send_sem=sA_s.at[c], recv_sem=sA_r.at[c],
                device_id=px, device_id_type=pl.DeviceIdType.MESH,
            )

        def rdma_b(c, dst_buf, ss, rs, dev):
            return pltpu.make_async_remote_copy(
                src_ref=rsend.at[c], dst_ref=dst_buf.at[c],
                send_sem=ss.at[c], recv_sem=rs.at[c],
                device_id=dev, device_id_type=pl.DeviceIdType.MESH,
            )

        def finish_a(c):
            rdma_a(c).wait()
            r = precv[c].astype(jnp.float32) + psend[c].astype(jnp.float32)
            out_ref[pl.ds(q * BLK, BLK), c * CD:(c + 1) * CD] = r
            rsend[c] = r.astype(jnp.bfloat16)
            rdma_b(c, rcv_y, sY_s, sY_r, py).start()
            rdma_b(c, rcv_z, sZ_s, sZ_r, pz).start()
            rdma_b(c, rcv_d, sD_s, sD_r, pd).start()

        for c in range(NC):
            if c + 1 < NC:
                w_copy(c + 1).start()
            w_copy(c).wait()
            p = lax.dot_general(
                a_ref[...], wbuf[c % 2].astype(jnp.bfloat16),
                (((1,), (1,)), ((), ())),
                preferred_element_type=jnp.float32,
            )
            psend[c] = p.astype(jnp.bfloat16)
            rdma_a(c).start()
            if c >= 1:
                finish_a(c - 1)
        finish_a(NC - 1)

        for c in range(NC):
            rdma_b(c, rcv_y, sY_s, sY_r, py).wait()
            out_ref[pl.ds(qy * BLK, BLK), c * CD:(c + 1) * CD] = (
                rcv_y[c].astype(jnp.float32)
            )
            rdma_b(c, rcv_z, sZ_s, sZ_r, pz).wait()
            out_ref[pl.ds(qz * BLK, BLK), c * CD:(c + 1) * CD] = (
                rcv_z[c].astype(jnp.float32)
            )
            rdma_b(c, rcv_d, sD_s, sD_r, pd).wait()
            out_ref[pl.ds(qd * BLK, BLK), c * CD:(c + 1) * CD] = (
                rcv_d[c].astype(jnp.float32)
            )

    return pl.pallas_call(
        body,
        out_shape=jax.ShapeDtypeStruct((m, d), jnp.float32),
        in_specs=[
            pl.BlockSpec(memory_space=pl.ANY),
            pl.BlockSpec(memory_space=pl.ANY),
        ],
        out_specs=pl.BlockSpec(memory_space=pltpu.VMEM),
        scratch_shapes=[
            pltpu.VMEM((BLK, f), jnp.float32),
            pltpu.VMEM((BLK, f), jnp.bfloat16),
            pltpu.VMEM((2, CD, f), jnp.float32),
            pltpu.VMEM((NC, BLK, CD), jnp.bfloat16),
            pltpu.VMEM((NC, BLK, CD), jnp.bfloat16),
            pltpu.VMEM((NC, BLK, CD), jnp.bfloat16),
            pltpu.VMEM((NC, BLK, CD), jnp.bfloat16),
            pltpu.VMEM((NC, BLK, CD), jnp.bfloat16),
            pltpu.VMEM((NC, BLK, CD), jnp.bfloat16),
            pltpu.SemaphoreType.DMA,
            pltpu.SemaphoreType.DMA((2,)),
            pltpu.SemaphoreType.DMA((NC,)),
            pltpu.SemaphoreType.DMA((NC,)),
            pltpu.SemaphoreType.DMA((NC,)),
            pltpu.SemaphoreType.DMA((NC,)),
            pltpu.SemaphoreType.DMA((NC,)),
            pltpu.SemaphoreType.DMA((NC,)),
            pltpu.SemaphoreType.DMA((NC,)),
            pltpu.SemaphoreType.DMA((NC,)),
        ],
        compiler_params=pltpu.CompilerParams(collective_id=0),
    )(dy, W)


# device time: 31406 ns/iter; 1.0219x vs baseline; 1.0219x over previous
import jax
import jax.numpy as jnp
from jax import lax
from jax.experimental import pallas as pl
from jax.experimental.pallas import tpu as pltpu

BLK = 256
NC = 4
CD = 1024 // NC


def kernel(dy, W):
    m, f = dy.shape
    d = W.shape[0]

    def body(
        dy_hbm, w_hbm, out_ref,
        dyb, a_ref, wbuf, psend, precv,
        dy_sem, w_sems, sA_s, sA_r, sY_s, sY_r, sZ_s, sZ_r, sD_s, sD_r,
    ):
        my_x = lax.axis_index("x")
        my_y = lax.axis_index("y")
        my_z = lax.axis_index("z")
        px = (1 - my_x, my_y, my_z)
        py = (my_x, 1 - my_y, my_z)
        pz = (my_x, my_y, 1 - my_z)
        pd = (my_x, 1 - my_y, 1 - my_z)

        q = 2 * my_y + my_z

        barrier = pltpu.get_barrier_semaphore()
        for p in (px, py, pz, pd):
            pl.semaphore_signal(
                barrier, inc=1, device_id=p,
                device_id_type=pl.DeviceIdType.MESH,
            )
        pl.semaphore_wait(barrier, 4)

        dy_cp = pltpu.make_async_copy(
            dy_hbm.at[pl.ds(q * BLK, BLK), :], dyb, dy_sem
        )
        dy_cp.start()

        def w_copy(c):
            return pltpu.make_async_copy(
                w_hbm.at[pl.ds(c * CD, CD), :], wbuf.at[c % 2],
                w_sems.at[c % 2],
            )

        w_copy(0).start()
        dy_cp.wait()
        a_ref[...] = dyb[...].astype(jnp.bfloat16)

        def rdma_a(c):
            return pltpu.make_async_remote_copy(
                src_ref=psend.at[c], dst_ref=precv.at[c],
                send_sem=sA_s.at[c], recv_sem=sA_r.at[c],
                device_id=px, device_id_type=pl.DeviceIdType.MESH,
            )

        def rdma_b(c, ss, rs, dev):
            blk = out_ref.at[pl.ds(q * BLK, BLK), pl.ds(c * CD, CD)]
            return pltpu.make_async_remote_copy(
                src_ref=blk, dst_ref=blk,
                send_sem=ss.at[c], recv_sem=rs.at[c],
                device_id=dev, device_id_type=pl.DeviceIdType.MESH,
            )

        def finish_a(c):
            rdma_a(c).wait()
            out_ref[pl.ds(q * BLK, BLK), c * CD:(c + 1) * CD] = (
                precv[c] + psend[c]
            )
            rdma_b(c, sY_s, sY_r, py).start()
            rdma_b(c, sZ_s, sZ_r, pz).start()
            rdma_b(c, sD_s, sD_r, pd).start()

        for c in range(NC):
            if c + 1 < NC:
                w_copy(c + 1).start()
            w_copy(c).wait()
            p = lax.dot_general(
                a_ref[...], wbuf[c % 2].astype(jnp.bfloat16),
                (((1,), (1,)), ((), ())),
                preferred_element_type=jnp.float32,
            )
            psend[c] = p.astype(jnp.bfloat16)
            rdma_a(c).start()
            if c >= 1:
                finish_a(c - 1)
        finish_a(NC - 1)

        for c in range(NC):
            rdma_b(c, sY_s, sY_r, py).wait()
            rdma_b(c, sZ_s, sZ_r, pz).wait()
            rdma_b(c, sD_s, sD_r, pd).wait()

    return pl.pallas_call(
        body,
        out_shape=jax.ShapeDtypeStruct((m, d), jnp.bfloat16),
        in_specs=[
            pl.BlockSpec(memory_space=pl.ANY),
            pl.BlockSpec(memory_space=pl.ANY),
        ],
        out_specs=pl.BlockSpec(memory_space=pltpu.VMEM),
        scratch_shapes=[
            pltpu.VMEM((BLK, f), jnp.float32),
            pltpu.VMEM((BLK, f), jnp.bfloat16),
            pltpu.VMEM((2, CD, f), jnp.float32),
            pltpu.VMEM((NC, BLK, CD), jnp.bfloat16),
            pltpu.VMEM((NC, BLK, CD), jnp.bfloat16),
            pltpu.SemaphoreType.DMA,
            pltpu.SemaphoreType.DMA((2,)),
            pltpu.SemaphoreType.DMA((NC,)),
            pltpu.SemaphoreType.DMA((NC,)),
            pltpu.SemaphoreType.DMA((NC,)),
            pltpu.SemaphoreType.DMA((NC,)),
            pltpu.SemaphoreType.DMA((NC,)),
            pltpu.SemaphoreType.DMA((NC,)),
            pltpu.SemaphoreType.DMA((NC,)),
            pltpu.SemaphoreType.DMA((NC,)),
        ],
        compiler_params=pltpu.CompilerParams(collective_id=0),
    )(dy, W)
